# baseline (speedup 1.0000x reference)
import functools

import jax
import jax.numpy as jnp
from jax import lax
from jax.experimental import pallas as pl
from jax.experimental.pallas import tpu as pltpu
from jax.experimental.pallas import tpu_sc as plsc

_NC = 2
_NS = 16
_NW = _NC * _NS
_CHUNK = 128
_K = 16
_BLOCK = _K * _CHUNK
_WA = 8
_WB = 16


def _cdiv(a, b):
    return (a + b - 1) // b


def _make_sc_pass(n_slop, nb, w, stage_tbl):
    assert nb % 2 == 0
    rpt = n_slop // _NS
    mesh = plsc.VectorSubcoreMesh(
        core_axis_name="c", subcore_axis_name="s",
        num_cores=_NC, num_subcores=_NS)

    @functools.partial(
        pl.kernel,
        out_type=jax.ShapeDtypeStruct((_NC, n_slop, w), jnp.float32),
        mesh=mesh,
        scratch_types=[
            pltpu.VMEM((2, _K, _CHUNK), jnp.int32),
            pltpu.VMEM((2, _K, _CHUNK), jnp.int32),
            pltpu.VMEM((2, _BLOCK, w), jnp.float32),
            pltpu.VMEM_SHARED((n_slop, w), jnp.float32),
            pltpu.VMEM_SHARED((n_slop if stage_tbl else 1, w), jnp.float32),
            pltpu.SemaphoreType.DMA,
            pltpu.SemaphoreType.DMA,
            pltpu.SemaphoreType.DMA,
            pltpu.SemaphoreType.DMA,
        ],
        compiler_params=pltpu.CompilerParams(use_tc_tiling_on_sc=False),
    )
    def sc_pass(table, edges, zeros, out, src_v, dst_v, rows_v, acc,
                tbl, gsem0, gsem1, ssem0, ssem1):
        c = lax.axis_index("c")
        s = lax.axis_index("s")
        wid = s * _NC + c
        gsem = (gsem0, gsem1)
        ssem = (ssem0, ssem1)
        pltpu.sync_copy(zeros, acc.at[pl.ds(s * rpt, rpt)])
        if stage_tbl:
            pltpu.sync_copy(table.at[pl.ds(s * rpt, rpt)],
                            tbl.at[pl.ds(s * rpt, rpt)])
        gsrc = tbl if stage_tbl else table
        plsc.subcore_barrier()

        def load_and_fire(b, p):
            pltpu.sync_copy(edges.at[0, wid, b], src_v.at[p])
            pltpu.sync_copy(edges.at[1, wid, b], dst_v.at[p])
            for j in range(_K):
                pltpu.async_copy(gsrc.at[src_v.at[p, j]],
                                 rows_v.at[p, pl.ds(j * _CHUNK, _CHUNK)],
                                 gsem[p])

        def wait_gathers(p):
            for j in range(_K):
                pltpu.make_async_copy(
                    gsrc.at[src_v.at[p, j]],
                    rows_v.at[p, pl.ds(j * _CHUNK, _CHUNK)],
                    gsem[p]).wait()

        def fire_scatters(p):
            for j in range(_K):
                pltpu.async_copy(rows_v.at[p, pl.ds(j * _CHUNK, _CHUNK)],
                                 acc.at[dst_v.at[p, j]], ssem[p], add=True)

        def wait_scatters(p):
            for j in range(_K):
                pltpu.make_async_copy(
                    rows_v.at[p, pl.ds(j * _CHUNK, _CHUNK)],
                    acc.at[dst_v.at[p, j]], ssem[p]).wait()

        load_and_fire(0, 0)

        def blk2(b2, carry):
            a = 2 * b2
            @pl.when(b2 > 0)
            def _():
                wait_scatters(1)
            load_and_fire(a + 1, 1)
            wait_gathers(0)
            fire_scatters(0)
            wait_scatters(0)
            @pl.when(b2 + 1 < nb // 2)
            def _():
                load_and_fire(a + 2, 0)
            wait_gathers(1)
            fire_scatters(1)
            return carry

        lax.fori_loop(0, nb // 2, blk2, 0)
        wait_scatters(1)
        plsc.subcore_barrier()
        pltpu.sync_copy(acc.at[pl.ds(s * rpt, rpt)],
                        out.at[c, pl.ds(s * rpt, rpt)])

    return sc_pass


def _bd(a):
    return jnp.kron(jnp.eye(16, dtype=jnp.float32), a)


def _tc1(xa, pos_p, A1, Adeg16, Ap1, A2, Adeg32, Ap2, Adeg0,
         s1L, t1L, bnr):
    n = xa.shape[1]
    grid = n // bnr
    f32 = jnp.float32

    def body(xr, posr, a1r, ad16r, ap1r, a2r, ad32r, ap2r, ad0r,
             s1r, t1r, out1r, c2r, invr):
        xb = xr[...]
        S = xb[0] + xb[1]
        posb = posr[...]
        pre = jnp.dot(S, a1r[...], preferred_element_type=f32,
                      precision=jax.lax.Precision.HIGHEST)
        degr = jnp.dot(S, ad16r[...], preferred_element_type=f32,
                      precision=jax.lax.Precision.HIGHEST)
        posw1 = jnp.dot(posb, ap1r[...], preferred_element_type=f32,
                      precision=jax.lax.Precision.HIGHEST)
        iv16 = 1.0 / jnp.maximum(degr, 1.0)
        agg1 = pre - degr * posw1
        out1r[...] = jnp.maximum(agg1 * iv16 * s1r[...] + t1r[...], 0.0)
        pre2 = jnp.dot(S, a2r[...], preferred_element_type=f32,
                      precision=jax.lax.Precision.HIGHEST)
        deg32 = jnp.dot(S, ad32r[...], preferred_element_type=f32,
                      precision=jax.lax.Precision.HIGHEST)
        posw2 = jnp.dot(posb, ap2r[...], preferred_element_type=f32,
                      precision=jax.lax.Precision.HIGHEST)
        c2r[...] = pre2 - deg32 * posw2
        invr[...] = 1.0 / jnp.maximum(
            jnp.dot(S, ad0r[...], preferred_element_type=f32,
                      precision=jax.lax.Precision.HIGHEST), 1.0)

    cst = lambda i: (0, 0)
    return pl.pallas_call(
        body,
        grid=(grid,),
        in_specs=[
            pl.BlockSpec((2, bnr, 128), lambda i: (0, i, 0)),
            pl.BlockSpec((bnr, 128), lambda i: (i, 0)),
            pl.BlockSpec((128, 256), cst),
            pl.BlockSpec((128, 256), cst),
            pl.BlockSpec((128, 256), cst),
            pl.BlockSpec((128, 512), cst),
            pl.BlockSpec((128, 512), cst),
            pl.BlockSpec((128, 512), cst),
            pl.BlockSpec((128, 128), cst),
            pl.BlockSpec((1, 256), cst),
            pl.BlockSpec((1, 256), cst),
        ],
        out_specs=[
            pl.BlockSpec((bnr, 256), lambda i: (i, 0)),
            pl.BlockSpec((bnr, 512), lambda i: (i, 0)),
            pl.BlockSpec((bnr, 128), lambda i: (i, 0)),
        ],
        out_shape=[
            jax.ShapeDtypeStruct((n, 256), f32),
            jax.ShapeDtypeStruct((n, 512), f32),
            jax.ShapeDtypeStruct((n, 128), f32),
        ],
    )(xa, pos_p, A1, Adeg16, Ap1, A2, Adeg32, Ap2, Adeg0, s1L, t1L)


def _tc2(xb, C2p, invc, BDF, Arep, s2L, t2L, bnr, n_out):
    n = xb.shape[1]
    grid = n // bnr
    f32 = jnp.float32

    def body(xr, c2r, invr, bdfr, arepr, s2r, t2r, outr):
        xb_ = xr[...]
        Sf = xb_[0] + xb_[1]
        Z = jnp.dot(Sf, bdfr[...], preferred_element_type=f32,
                      precision=jax.lax.Precision.HIGHEST)
        iv = jnp.dot(invr[...], arepr[...], preferred_element_type=f32,
                      precision=jax.lax.Precision.HIGHEST)
        outr[...] = jnp.maximum(
            (Z + c2r[...]) * iv * s2r[...] + t2r[...], 0.0)

    cst = lambda i: (0, 0)
    return pl.pallas_call(
        body,
        grid=(grid,),
        in_specs=[
            pl.BlockSpec((2, bnr, 256), lambda i: (0, i, 0)),
            pl.BlockSpec((bnr, 512), lambda i: (i, 0)),
            pl.BlockSpec((bnr, 128), lambda i: (i, 0)),
            pl.BlockSpec((256, 512), cst),
            pl.BlockSpec((128, 512), cst),
            pl.BlockSpec((1, 512), cst),
            pl.BlockSpec((1, 512), cst),
        ],
        out_specs=pl.BlockSpec((bnr, 512), lambda i: (i, 0)),
        out_shape=jax.ShapeDtypeStruct((n, 512), f32),
    )(xb, C2p, invc, BDF, Arep, s2L, t2L)


def kernel(x, pos, edge_index, W1, b1, s1, t1, W2, b2, s2, t2):
    N = x.shape[0]
    E = edge_index.shape[1]
    n_slop = _cdiv(N + 1, _NS * 8) * _NS * 8
    rpt = n_slop // _NS
    nb = _cdiv(E, _NW * _BLOCK)
    nb += nb % 2
    e_pad = nb * _NW * _BLOCK
    pad = e_pad - E

    pad_i = jnp.arange(pad, dtype=jnp.int32)
    slop = n_slop - N
    edges = jnp.concatenate(
        [edge_index, jnp.stack([pad_i % N, N + pad_i % slop])],
        axis=1).reshape(2, _NW, nb, _K, _CHUNK)
    zerosA = jnp.zeros((rpt, _WA), jnp.float32)
    zerosB = jnp.zeros((rpt, _WB), jnp.float32)

    table1 = jnp.pad(jnp.concatenate(
        [x, pos, jnp.ones((N, 1), jnp.float32),
         jnp.zeros((N, _WA - 5), jnp.float32)], axis=1),
        ((0, n_slop - N), (0, 0)))

    pA = _make_sc_pass(n_slop, nb, _WA, False)(table1, edges, zerosA)

    nr = n_slop // 16
    XA = jnp.maximum(pA, 0.0).reshape(_NC, nr, 128)
    pos_p = jnp.pad(pos, ((0, n_slop - N), (0, 5))).reshape(nr, 128)

    z316 = jnp.zeros((3, 16), jnp.float32)
    A1 = _bd(jnp.concatenate([W1.T, b1[None, :], z316], axis=0))
    Adeg16 = _bd(jnp.concatenate(
        [jnp.zeros((4, 16), jnp.float32), jnp.ones((1, 16), jnp.float32),
         z316], axis=0))
    Ap1 = _bd(jnp.concatenate(
        [W1[:, 1:4].T, jnp.zeros((5, 16), jnp.float32)], axis=0))
    z332 = jnp.zeros((3, 32), jnp.float32)
    A2 = _bd(jnp.concatenate(
        [jnp.zeros((1, 32), jnp.float32), W2[:, 16:19].T, b2[None, :],
         z332], axis=0))
    Adeg32 = _bd(jnp.concatenate(
        [jnp.zeros((4, 32), jnp.float32), jnp.ones((1, 32), jnp.float32),
         z332], axis=0))
    Ap2 = _bd(jnp.concatenate(
        [W2[:, 16:19].T, jnp.zeros((5, 32), jnp.float32)], axis=0))
    e0 = jnp.zeros((8, 8), jnp.float32).at[4, 0].set(1.0)
    Adeg0 = _bd(e0)
    s1L = jnp.tile(s1, 16)[None, :]
    t1L = jnp.tile(t1, 16)[None, :]

    bnr = 184
    out1p, C2p, invc = _tc1(XA, pos_p, A1, Adeg16, Ap1,
                            A2, Adeg32, Ap2, Adeg0, s1L, t1L, bnr)

    out1 = out1p.reshape(n_slop, 16)
    pB = _make_sc_pass(n_slop, nb, _WB, False)(out1, edges, zerosB)

    XB = jnp.maximum(pB, 0.0).reshape(_NC, nr, 256)
    BDF = jnp.kron(jnp.eye(16, dtype=jnp.float32), W2[:, :16].T)
    Arep = _bd(jnp.zeros((8, 32), jnp.float32).at[0, :].set(1.0))
    s2L = jnp.tile(s2, 16)[None, :]
    t2L = jnp.tile(t2, 16)[None, :]
    out2p = _tc2(XB, C2p, invc, BDF, Arep, s2L, t2L, bnr, N)
    return jnp.maximum(out2p.reshape(n_slop, 32)[:N], 0.0)


# speedup vs baseline: 1.0930x; 1.0930x over previous
import functools

import jax
import jax.numpy as jnp
from jax import lax
from jax.experimental import pallas as pl
from jax.experimental.pallas import tpu as pltpu
from jax.experimental.pallas import tpu_sc as plsc

_NC = 2
_NS = 16
_NW = _NC * _NS
_CHUNK = 128
_K = 16
_BLOCK = _K * _CHUNK
_WA = 8
_WB = 16


def _cdiv(a, b):
    return (a + b - 1) // b


def _make_sc_pass(n_slop, nb, w, stage_tbl):
    assert nb % 2 == 0
    rpt = n_slop // _NS
    mesh = plsc.VectorSubcoreMesh(
        core_axis_name="c", subcore_axis_name="s",
        num_cores=_NC, num_subcores=_NS)

    @functools.partial(
        pl.kernel,
        out_type=jax.ShapeDtypeStruct((_NC, n_slop, w), jnp.float32),
        mesh=mesh,
        scratch_types=[
            pltpu.VMEM((2, _K, _CHUNK), jnp.int32),
            pltpu.VMEM((2, _K, _CHUNK), jnp.int32),
            pltpu.VMEM((2, _BLOCK, w), jnp.float32),
            pltpu.VMEM_SHARED((n_slop, w), jnp.float32),
            pltpu.VMEM_SHARED((n_slop if stage_tbl else 1, w), jnp.float32),
            pltpu.SemaphoreType.DMA,
            pltpu.SemaphoreType.DMA,
            pltpu.SemaphoreType.DMA,
            pltpu.SemaphoreType.DMA,
        ],
        compiler_params=pltpu.CompilerParams(use_tc_tiling_on_sc=False),
    )
    def sc_pass(table, edges, zeros, out, src_v, dst_v, rows_v, acc,
                tbl, gsem0, gsem1, ssem0, ssem1):
        c = lax.axis_index("c")
        s = lax.axis_index("s")
        wid = s * _NC + c
        gsem = (gsem0, gsem1)
        ssem = (ssem0, ssem1)
        pltpu.sync_copy(zeros, acc.at[pl.ds(s * rpt, rpt)])
        if stage_tbl:
            pltpu.sync_copy(table.at[pl.ds(s * rpt, rpt)],
                            tbl.at[pl.ds(s * rpt, rpt)])
        gsrc = tbl if stage_tbl else table
        plsc.subcore_barrier()

        def load_and_fire(b, p):
            pltpu.sync_copy(edges.at[0, wid, b], src_v.at[p])
            pltpu.sync_copy(edges.at[1, wid, b], dst_v.at[p])
            for j in range(_K):
                pltpu.async_copy(gsrc.at[src_v.at[p, j]],
                                 rows_v.at[p, pl.ds(j * _CHUNK, _CHUNK)],
                                 gsem[p])

        def wait_gathers(p):
            for j in range(_K):
                pltpu.make_async_copy(
                    gsrc.at[src_v.at[p, j]],
                    rows_v.at[p, pl.ds(j * _CHUNK, _CHUNK)],
                    gsem[p]).wait()

        def fire_scatters(p):
            for j in range(_K):
                pltpu.async_copy(rows_v.at[p, pl.ds(j * _CHUNK, _CHUNK)],
                                 acc.at[dst_v.at[p, j]], ssem[p], add=True)

        def wait_scatters(p):
            for j in range(_K):
                pltpu.make_async_copy(
                    rows_v.at[p, pl.ds(j * _CHUNK, _CHUNK)],
                    acc.at[dst_v.at[p, j]], ssem[p]).wait()

        load_and_fire(0, 0)

        def blk2(b2, carry):
            a = 2 * b2
            @pl.when(b2 > 0)
            def _():
                wait_scatters(1)
            load_and_fire(a + 1, 1)
            wait_gathers(0)
            fire_scatters(0)
            wait_scatters(0)
            @pl.when(b2 + 1 < nb // 2)
            def _():
                load_and_fire(a + 2, 0)
            wait_gathers(1)
            fire_scatters(1)
            return carry

        lax.fori_loop(0, nb // 2, blk2, 0)
        wait_scatters(1)
        plsc.subcore_barrier()
        pltpu.sync_copy(acc.at[pl.ds(s * rpt, rpt)],
                        out.at[c, pl.ds(s * rpt, rpt)])

    return sc_pass


def _bd(a):
    return jnp.kron(jnp.eye(16, dtype=jnp.float32), a)


def _tc1(xa0, xa1, pos_p, A1, Adeg16, Ap1, A2, Adeg32, Ap2, Adeg0,
         s1L, t1L, bnr):
    n = xa0.shape[0]
    grid = n // bnr
    f32 = jnp.float32

    def body(x0r, x1r, posr, a1r, ad16r, ap1r, a2r, ad32r, ap2r, ad0r,
             s1r, t1r, out1r, c2r, invr):
        S = x0r[...] + x1r[...]
        posb = posr[...]
        pre = jnp.dot(S, a1r[...], preferred_element_type=f32,
                      precision=jax.lax.Precision.HIGHEST)
        degr = jnp.dot(S, ad16r[...], preferred_element_type=f32,
                      precision=jax.lax.Precision.HIGHEST)
        posw1 = jnp.dot(posb, ap1r[...], preferred_element_type=f32,
                      precision=jax.lax.Precision.HIGHEST)
        iv16 = 1.0 / jnp.maximum(degr, 1.0)
        agg1 = pre - degr * posw1
        out1r[...] = jnp.maximum(agg1 * iv16 * s1r[...] + t1r[...], 0.0)
        pre2 = jnp.dot(S, a2r[...], preferred_element_type=f32,
                      precision=jax.lax.Precision.HIGHEST)
        deg32 = jnp.dot(S, ad32r[...], preferred_element_type=f32,
                      precision=jax.lax.Precision.HIGHEST)
        posw2 = jnp.dot(posb, ap2r[...], preferred_element_type=f32,
                      precision=jax.lax.Precision.HIGHEST)
        c2r[...] = pre2 - deg32 * posw2
        invr[...] = 1.0 / jnp.maximum(
            jnp.dot(S, ad0r[...], preferred_element_type=f32,
                      precision=jax.lax.Precision.HIGHEST), 1.0)

    cst = lambda i: (0, 0)
    return pl.pallas_call(
        body,
        grid=(grid,),
        in_specs=[
            pl.BlockSpec((bnr, 128), lambda i: (i, 0)),
            pl.BlockSpec((bnr, 128), lambda i: (i, 0)),
            pl.BlockSpec((bnr, 128), lambda i: (i, 0)),
            pl.BlockSpec((128, 256), cst),
            pl.BlockSpec((128, 256), cst),
            pl.BlockSpec((128, 256), cst),
            pl.BlockSpec((128, 512), cst),
            pl.BlockSpec((128, 512), cst),
            pl.BlockSpec((128, 512), cst),
            pl.BlockSpec((128, 128), cst),
            pl.BlockSpec((1, 256), cst),
            pl.BlockSpec((1, 256), cst),
        ],
        out_specs=[
            pl.BlockSpec((bnr, 256), lambda i: (i, 0)),
            pl.BlockSpec((bnr, 512), lambda i: (i, 0)),
            pl.BlockSpec((bnr, 128), lambda i: (i, 0)),
        ],
        out_shape=[
            jax.ShapeDtypeStruct((n, 256), f32),
            jax.ShapeDtypeStruct((n, 512), f32),
            jax.ShapeDtypeStruct((n, 128), f32),
        ],
    )(xa0, xa1, pos_p, A1, Adeg16, Ap1, A2, Adeg32, Ap2, Adeg0, s1L, t1L)


def _tc2(xb0, xb1, C2p, invc, BDF, Arep, s2L, t2L, bnr, n_out):
    n = xb0.shape[0]
    grid = n // bnr
    f32 = jnp.float32

    def body(x0r, x1r, c2r, invr, bdfr, arepr, s2r, t2r, outr):
        Sf = x0r[...] + x1r[...]
        Z = jnp.dot(Sf, bdfr[...], preferred_element_type=f32,
                      precision=jax.lax.Precision.HIGHEST)
        iv = jnp.dot(invr[...], arepr[...], preferred_element_type=f32,
                      precision=jax.lax.Precision.HIGHEST)
        outr[...] = jnp.maximum(
            (Z + c2r[...]) * iv * s2r[...] + t2r[...], 0.0)

    cst = lambda i: (0, 0)
    return pl.pallas_call(
        body,
        grid=(grid,),
        in_specs=[
            pl.BlockSpec((bnr, 256), lambda i: (i, 0)),
            pl.BlockSpec((bnr, 256), lambda i: (i, 0)),
            pl.BlockSpec((bnr, 512), lambda i: (i, 0)),
            pl.BlockSpec((bnr, 128), lambda i: (i, 0)),
            pl.BlockSpec((256, 512), cst),
            pl.BlockSpec((128, 512), cst),
            pl.BlockSpec((1, 512), cst),
            pl.BlockSpec((1, 512), cst),
        ],
        out_specs=pl.BlockSpec((bnr, 512), lambda i: (i, 0)),
        out_shape=jax.ShapeDtypeStruct((n, 512), f32),
    )(xb0, xb1, C2p, invc, BDF, Arep, s2L, t2L)


def kernel(x, pos, edge_index, W1, b1, s1, t1, W2, b2, s2, t2):
    N = x.shape[0]
    E = edge_index.shape[1]
    n_slop = _cdiv(N + 1, _NS * 8) * _NS * 8
    rpt = n_slop // _NS
    nb = _cdiv(E, _NW * _BLOCK)
    nb += nb % 2
    e_pad = nb * _NW * _BLOCK
    pad = e_pad - E

    pad_i = jnp.arange(pad, dtype=jnp.int32)
    slop = n_slop - N
    edges = jnp.concatenate(
        [edge_index, jnp.stack([pad_i % N, N + pad_i % slop])],
        axis=1).reshape(2, _NW, nb, _K, _CHUNK)
    zerosA = jnp.zeros((rpt, _WA), jnp.float32)
    zerosB = jnp.zeros((rpt, _WB), jnp.float32)

    table1 = jnp.pad(jnp.concatenate(
        [x, pos, jnp.ones((N, 1), jnp.float32),
         jnp.zeros((N, _WA - 5), jnp.float32)], axis=1),
        ((0, n_slop - N), (0, 0)))

    pA = _make_sc_pass(n_slop, nb, _WA, False)(table1, edges, zerosA)

    nr = n_slop // 16
    XA = pA.reshape(_NC, nr, 128)
    pos_p = jnp.pad(pos, ((0, n_slop - N), (0, 5))).reshape(nr, 128)

    z316 = jnp.zeros((3, 16), jnp.float32)
    A1 = _bd(jnp.concatenate([W1.T, b1[None, :], z316], axis=0))
    Adeg16 = _bd(jnp.concatenate(
        [jnp.zeros((4, 16), jnp.float32), jnp.ones((1, 16), jnp.float32),
         z316], axis=0))
    Ap1 = _bd(jnp.concatenate(
        [W1[:, 1:4].T, jnp.zeros((5, 16), jnp.float32)], axis=0))
    z332 = jnp.zeros((3, 32), jnp.float32)
    A2 = _bd(jnp.concatenate(
        [jnp.zeros((1, 32), jnp.float32), W2[:, 16:19].T, b2[None, :],
         z332], axis=0))
    Adeg32 = _bd(jnp.concatenate(
        [jnp.zeros((4, 32), jnp.float32), jnp.ones((1, 32), jnp.float32),
         z332], axis=0))
    Ap2 = _bd(jnp.concatenate(
        [W2[:, 16:19].T, jnp.zeros((5, 32), jnp.float32)], axis=0))
    e0 = jnp.zeros((8, 8), jnp.float32).at[4, 0].set(1.0)
    Adeg0 = _bd(e0)
    s1L = jnp.tile(s1, 16)[None, :]
    t1L = jnp.tile(t1, 16)[None, :]

    bnr = 184
    out1p, C2p, invc = _tc1(XA[0], XA[1], pos_p, A1, Adeg16, Ap1,
                            A2, Adeg32, Ap2, Adeg0, s1L, t1L, bnr)

    out1 = out1p.reshape(n_slop, 16)
    pB = _make_sc_pass(n_slop, nb, _WB, False)(out1, edges, zerosB)

    XB = pB.reshape(_NC, nr, 256)
    BDF = jnp.kron(jnp.eye(16, dtype=jnp.float32), W2[:, :16].T)
    Arep = _bd(jnp.zeros((8, 32), jnp.float32).at[0, :].set(1.0))
    s2L = jnp.tile(s2, 16)[None, :]
    t2L = jnp.tile(t2, 16)[None, :]
    out2p = _tc2(XB[0], XB[1], C2p, invc, BDF, Arep, s2L, t2L, bnr, N)
    return jnp.maximum(out2p.reshape(n_slop, 32)[:N], 0.0)
